# baseline (speedup 1.0000x reference)
import functools

import jax
import jax.numpy as jnp
from jax import lax
from jax.experimental import pallas as pl
from jax.experimental.pallas import tpu as pltpu
from jax.experimental.pallas import tpu_sc as plsc

N = 9999
NPAD = 10240
E = 319968
EPAD = 327680
ROWS = EPAD // 128
D = 128
C = 2
BL = 912


@functools.lru_cache(maxsize=None)
def _mesh():
    return plsc.VectorSubcoreMesh(core_axis_name="c", subcore_axis_name="s")


def _deg_body(srcdst_hbm, out_hbm, buf, idxb, acc, sem_s0, sem_s1, sem_x):
    c = lax.axis_index("c")
    s = lax.axis_index("s")
    NT = 40
    sems = (sem_s0, sem_s1)

    def fill_zero(i, _):
        buf[i, :] = jnp.zeros((16,), jnp.float32)
        return 0

    lax.fori_loop(0, 512, fill_zero, 0)
    pltpu.sync_copy(buf, acc.at[pl.ds(s * 640, 512)])
    pltpu.sync_copy(buf.at[pl.ds(0, 128)], acc.at[pl.ds(s * 640 + 512, 128)])

    def fill_one(i, _):
        buf[i, :] = jnp.ones((16,), jnp.float32)
        return 0

    lax.fori_loop(0, 128, fill_one, 0)
    plsc.subcore_barrier()

    base = s * 160

    def issue_s(p, q):
        for j in range(4):
            pltpu.async_copy(buf.at[pl.ds(0, 128)], acc.at[idxb.at[q * 4 + j]],
                             sems[p], add=True)

    def wait_s(p, q):
        for j in range(4):
            pltpu.make_async_copy(buf.at[pl.ds(0, 128)],
                                  acc.at[idxb.at[q * 4 + j]], sems[p]).wait()

    def load_idx(q, r0):
        pltpu.async_copy(srcdst_hbm.at[c, pl.ds(r0, 4)],
                         idxb.at[pl.ds(q * 4, 4)], sem_x)

    def wait_idx(q):
        pltpu.make_async_copy(srcdst_hbm.at[c, pl.ds(0, 4)],
                              idxb.at[pl.ds(q * 4, 4)], sem_x).wait()

    pltpu.sync_copy(srcdst_hbm.at[c, pl.ds(base, 4)], idxb.at[pl.ds(0, 4)])

    def macro(m, _):
        for q in range(4):
            i = 4 * m + q

            @pl.when(i >= 2)
            def _():
                wait_s(q % 2, (q + 2) % 4)

            @pl.when(i >= 1)
            def _():
                wait_idx(q)

            issue_s(q % 2, q)

            @pl.when(i <= NT - 2)
            def _():
                load_idx((q + 1) % 4, base + (i + 1) * 4)

        return 0

    lax.fori_loop(0, NT // 4, macro, 0)
    wait_s((NT - 2) % 2, (NT - 2) % 4)
    wait_s((NT - 1) % 2, (NT - 1) % 4)
    plsc.subcore_barrier()
    pltpu.sync_copy(acc.at[pl.ds(s * 640, 640)], out_hbm.at[c, pl.ds(s * 640, 640)])


@functools.lru_cache(maxsize=None)
def _deg_kernel():
    return functools.partial(
        pl.kernel,
        compiler_params=pltpu.CompilerParams(use_tc_tiling_on_sc=False),
        out_type=jax.ShapeDtypeStruct((2, NPAD, 16), jnp.float32),
        mesh=_mesh(),
        scratch_types=[
            pltpu.VMEM((512, 16), jnp.float32),
            pltpu.VMEM((16, 128), jnp.int32),
            pltpu.VMEM_SHARED((NPAD, 16), jnp.float32),
            pltpu.SemaphoreType.DMA,
            pltpu.SemaphoreType.DMA,
            pltpu.SemaphoreType.DMA,
        ],
    )(_deg_body)


def _make_agg(width, nstreams, edge_split):
    def body(tab_hbm, src_hbm, dst_hbm, out_hbm, buf, sidx, didx, acc,
             g0, g1, g2, g3, s0, s1, s2, s3, sem_x):
        c = lax.axis_index("c")
        s = lax.axis_index("s")
        nv = width // 16
        gsems = (g0, g1, g2, g3)
        ssems = (s0, s1, s2, s3)

        def fill_zero(i, _):
            for j in range(nv):
                buf[i, pl.ds(j * 16, 16)] = jnp.zeros((16,), jnp.float32)
            return 0

        lax.fori_loop(0, 512, fill_zero, 0)
        pltpu.sync_copy(buf.at[pl.ds(0, 512)], acc.at[pl.ds(s * 640, 512)])
        pltpu.sync_copy(buf.at[pl.ds(0, 128)], acc.at[pl.ds(s * 640 + 512, 128)])
        plsc.subcore_barrier()

        if edge_split:
            base = c * (nstreams * 16) + s * nstreams
        else:
            base = s * nstreams

        def src_slice(r0, n):
            if edge_split:
                return src_hbm.at[pl.ds(r0, n)]
            return src_hbm.at[c, pl.ds(r0, n)]

        def bufsl(sl):
            return buf.at[pl.ds((sl % 8) * 128, 128)]

        def issue_g(q):
            pltpu.async_copy(tab_hbm.at[sidx.at[q % 16]], bufsl(q),
                             gsems[q % 4])

        def wait_g(q):
            pltpu.make_async_copy(tab_hbm.at[sidx.at[q % 16]], bufsl(q),
                                  gsems[q % 4]).wait()

        def issue_s(q):
            pltpu.async_copy(bufsl(q), acc.at[didx.at[q % 16]], ssems[q % 4],
                             add=True)

        def wait_s(q):
            pltpu.make_async_copy(bufsl(q), acc.at[didx.at[q % 16]],
                                  ssems[q % 4]).wait()

        def load_idx(ksl, r0):
            pltpu.async_copy(src_slice(r0, 4), sidx.at[pl.ds((ksl % 4) * 4, 4)],
                             sem_x)
            pltpu.async_copy(dst_hbm.at[pl.ds(r0, 4)],
                             didx.at[pl.ds((ksl % 4) * 4, 4)], sem_x)

        def wait_idx(ksl):
            pltpu.make_async_copy(src_slice(0, 4),
                                  sidx.at[pl.ds((ksl % 4) * 4, 4)], sem_x).wait()
            pltpu.make_async_copy(dst_hbm.at[pl.ds(0, 4)],
                                  didx.at[pl.ds((ksl % 4) * 4, 4)], sem_x).wait()

        pltpu.sync_copy(src_slice(base, 4), sidx.at[pl.ds(0, 4)])
        pltpu.sync_copy(dst_hbm.at[pl.ds(base, 4)], didx.at[pl.ds(0, 4)])
        load_idx(1, base + 4)
        for q in range(4):
            issue_g(q)

        def macro(m, _):
            for q in range(16):
                i = 16 * m + q
                wait_g(q)

                @pl.when(i >= 4)
                def _():
                    wait_s(q + 12)

                issue_s(q)

                if q % 4 == 0:
                    @pl.when(i <= nstreams - 5)
                    def _():
                        wait_idx(q // 4 + 1)

                    @pl.when(i <= nstreams - 9)
                    def _():
                        load_idx(q // 4 + 2, base + i + 8)

                @pl.when(i <= nstreams - 5)
                def _():
                    issue_g_late(q)

            return 0

        def issue_g_late(q):
            pltpu.async_copy(tab_hbm.at[sidx.at[(q + 4) % 16]], bufsl(q + 4),
                             gsems[q % 4])

        lax.fori_loop(0, nstreams // 16, macro, 0)
        for t in range(4):
            wait_s(12 + t)
        plsc.subcore_barrier()
        pltpu.sync_copy(acc.at[pl.ds(s * 640, 640)], out_hbm.at[c, pl.ds(s * 640, 640)])

    return functools.partial(
        pl.kernel,
        compiler_params=pltpu.CompilerParams(use_tc_tiling_on_sc=False),
        out_type=jax.ShapeDtypeStruct((2, NPAD, width), jnp.float32),
        mesh=_mesh(),
        scratch_types=[
            pltpu.VMEM((1024, width), jnp.float32),
            pltpu.VMEM((16, 128), jnp.int32),
            pltpu.VMEM((16, 128), jnp.int32),
            pltpu.VMEM_SHARED((NPAD, width), jnp.float32),
        ] + [pltpu.SemaphoreType.DMA] * 9,
    )(body)


@functools.lru_cache(maxsize=None)
def _aggc_kernel():
    return _make_agg(64, 160, edge_split=False)


@functools.lru_cache(maxsize=None)
def _agg16_kernel():
    return _make_agg(16, 80, edge_split=True)


def _agg128_body(tab_hbm, src_hbm, dst_hbm, out_hbm, buf, sidx, didx, acc,
                 sem_g, s0, s1, sem_x):
    c = lax.axis_index("c")
    s = lax.axis_index("s")
    nstreams = 80
    ssems = (s0, s1)

    def fill_zero(i, _):
        for j in range(8):
            buf[i, pl.ds(j * 16, 16)] = jnp.zeros((16,), jnp.float32)
        return 0

    lax.fori_loop(0, 256, fill_zero, 0)
    for t in range(2):
        pltpu.sync_copy(buf.at[pl.ds(0, 256)],
                        acc.at[pl.ds(s * 640 + t * 256, 256)])
    pltpu.sync_copy(buf.at[pl.ds(0, 128)], acc.at[pl.ds(s * 640 + 512, 128)])
    plsc.subcore_barrier()

    base = c * (nstreams * 16) + s * nstreams

    def bufsl(sl):
        return buf.at[pl.ds((sl % 2) * 128, 128)]

    def issue_g(q):
        pltpu.async_copy(tab_hbm.at[sidx.at[q % 16]], bufsl(q), sem_g)

    def wait_g(q):
        pltpu.make_async_copy(tab_hbm.at[sidx.at[q % 16]], bufsl(q),
                              sem_g).wait()

    def issue_s(q):
        pltpu.async_copy(bufsl(q), acc.at[didx.at[q % 16]], ssems[q % 2],
                         add=True)

    def wait_s(q):
        pltpu.make_async_copy(bufsl(q), acc.at[didx.at[q % 16]],
                              ssems[q % 2]).wait()

    def load_idx(ksl, r0):
        pltpu.async_copy(src_hbm.at[pl.ds(r0, 4)],
                         sidx.at[pl.ds((ksl % 4) * 4, 4)], sem_x)
        pltpu.async_copy(dst_hbm.at[pl.ds(r0, 4)],
                         didx.at[pl.ds((ksl % 4) * 4, 4)], sem_x)

    def wait_idx(ksl):
        pltpu.make_async_copy(src_hbm.at[pl.ds(0, 4)],
                              sidx.at[pl.ds((ksl % 4) * 4, 4)], sem_x).wait()
        pltpu.make_async_copy(dst_hbm.at[pl.ds(0, 4)],
                              didx.at[pl.ds((ksl % 4) * 4, 4)], sem_x).wait()

    pltpu.sync_copy(src_hbm.at[pl.ds(base, 4)], sidx.at[pl.ds(0, 4)])
    pltpu.sync_copy(dst_hbm.at[pl.ds(base, 4)], didx.at[pl.ds(0, 4)])
    load_idx(1, base + 4)
    issue_g(0)

    def issue_g_next(q):
        pltpu.async_copy(tab_hbm.at[sidx.at[(q + 1) % 16]], bufsl(q + 1),
                         sem_g)

    def macro(m, _):
        for q in range(16):
            i = 16 * m + q
            wait_g(q)
            issue_s(q)

            @pl.when(i >= 1)
            def _():
                wait_s(q + 15)

            if q % 4 == 0:
                @pl.when(i <= nstreams - 5)
                def _():
                    wait_idx(q // 4 + 1)

                @pl.when(i <= nstreams - 9)
                def _():
                    load_idx(q // 4 + 2, base + i + 8)

            @pl.when(i <= nstreams - 2)
            def _():
                issue_g_next(q)

        return 0

    lax.fori_loop(0, nstreams // 16, macro, 0)
    wait_s(15)
    plsc.subcore_barrier()
    pltpu.sync_copy(acc.at[pl.ds(s * 640, 640)], out_hbm.at[c, pl.ds(s * 640, 640)])


@functools.lru_cache(maxsize=None)
def _agg128_kernel():
    return functools.partial(
        pl.kernel,
        compiler_params=pltpu.CompilerParams(use_tc_tiling_on_sc=False),
        out_type=jax.ShapeDtypeStruct((2, NPAD, 128), jnp.float32),
        mesh=_mesh(),
        scratch_types=[
            pltpu.VMEM((256, 128), jnp.float32),
            pltpu.VMEM((16, 128), jnp.int32),
            pltpu.VMEM((16, 128), jnp.int32),
            pltpu.VMEM_SHARED((NPAD, 128), jnp.float32),
        ] + [pltpu.SemaphoreType.DMA] * 4,
    )(_agg128_body)


_BLK = 1024


def _prep_body(x_ref, dego_ref, o_ref):
    so = lax.rsqrt(jnp.clip(dego_ref[0][:, :1], 1.0, None))
    o_ref[...] = x_ref[...] * so


def _prep(x_pad, deg2):
    return pl.pallas_call(
        _prep_body,
        grid=(NPAD // _BLK,),
        in_specs=[
            pl.BlockSpec((_BLK, D), lambda i: (i, 0)),
            pl.BlockSpec((1, _BLK, 16), lambda i: (0, i, 0)),
        ],
        out_specs=pl.BlockSpec((_BLK, D), lambda i: (i, 0)),
        out_shape=jax.ShapeDtypeStruct((NPAD, D), jnp.float32),
    )(x_pad, deg2)


def _layer1_body(a0_ref, a1_ref, degi_ref, dego_ref, w_ref, b_ref, o_ref):
    si = lax.rsqrt(jnp.clip(degi_ref[0][:, :1], 1.0, None))
    so = lax.rsqrt(jnp.clip(dego_ref[0][:, :1], 1.0, None))
    a = (a0_ref[0] + a1_ref[0]) * si
    t = jnp.dot(a, w_ref[...], preferred_element_type=jnp.float32) + b_ref[...]
    o_ref[...] = jnp.maximum(t, 0.0) * so


def _layer2_body(a0_ref, a1_ref, degi_ref, dego_ref, w_ref, b_ref, w3_ref, o_ref):
    si = lax.rsqrt(jnp.clip(degi_ref[0][:, :1], 1.0, None))
    so = lax.rsqrt(jnp.clip(dego_ref[0][:, :1], 1.0, None))
    a = (a0_ref[0] + a1_ref[0]) * si
    t = jnp.dot(a, w_ref[...], preferred_element_type=jnp.float32) + b_ref[...]
    hn = jnp.tanh(t) * so
    o_ref[...] = jnp.dot(hn, w3_ref[...], preferred_element_type=jnp.float32)


def _layer1(agg, deg2, w, b):
    return pl.pallas_call(
        _layer1_body,
        grid=(NPAD // _BLK,),
        in_specs=[
            pl.BlockSpec((1, _BLK, D), lambda i: (0, i, 0)),
            pl.BlockSpec((1, _BLK, D), lambda i: (1, i, 0)),
            pl.BlockSpec((1, _BLK, 16), lambda i: (1, i, 0)),
            pl.BlockSpec((1, _BLK, 16), lambda i: (0, i, 0)),
            pl.BlockSpec((D, D), lambda i: (0, 0)),
            pl.BlockSpec((1, D), lambda i: (0, 0)),
        ],
        out_specs=pl.BlockSpec((_BLK, D), lambda i: (i, 0)),
        out_shape=jax.ShapeDtypeStruct((NPAD, D), jnp.float32),
    )(agg, agg, deg2, deg2, w, b)


def _layer2(agg, deg2, w, b, w3p):
    return pl.pallas_call(
        _layer2_body,
        grid=(NPAD // _BLK,),
        in_specs=[
            pl.BlockSpec((1, _BLK, D), lambda i: (0, i, 0)),
            pl.BlockSpec((1, _BLK, D), lambda i: (1, i, 0)),
            pl.BlockSpec((1, _BLK, 16), lambda i: (1, i, 0)),
            pl.BlockSpec((1, _BLK, 16), lambda i: (0, i, 0)),
            pl.BlockSpec((D, D), lambda i: (0, 0)),
            pl.BlockSpec((1, D), lambda i: (0, 0)),
            pl.BlockSpec((D, 16), lambda i: (0, 0)),
        ],
        out_specs=pl.BlockSpec((_BLK, 16), lambda i: (i, 0)),
        out_shape=jax.ShapeDtypeStruct((NPAD, 16), jnp.float32),
    )(agg, agg, deg2, deg2, w, b, w3p)


def _lstm_body(x_ref, wih_ref, whh_ref, bih_ref, bhh_ref, fw1_ref, fb1_ref,
               fw2_ref, fb2_ref, o_ref):
    b4 = bih_ref[...] + bhh_ref[...]
    wih = wih_ref[...]
    whh = whh_ref[...]
    dn = (((1,), (1,)), ((), ()))

    def step(t, hc):
        h, cc = hc
        xt = x_ref[t]
        g = (lax.dot_general(xt, wih, dn, preferred_element_type=jnp.float32)
             + lax.dot_general(h, whh, dn, preferred_element_type=jnp.float32)
             + b4)
        ig = jax.nn.sigmoid(g[:, 0:128])
        fg = jax.nn.sigmoid(g[:, 128:256])
        gg = jnp.tanh(g[:, 256:384])
        og = jax.nn.sigmoid(g[:, 384:512])
        cc = fg * cc + ig * gg
        return (og * jnp.tanh(cc), cc)

    h0 = jnp.zeros((BL, 128), jnp.float32)
    h, _ = lax.fori_loop(0, 11, step, (h0, h0))
    t1 = jnp.dot(h, fw1_ref[...], preferred_element_type=jnp.float32) + fb1_ref[...]
    o_ref[...] = jnp.dot(t1, fw2_ref[...], preferred_element_type=jnp.float32) + fb2_ref[...]


def _lstm(xT, Wih, Whh, bih, bhh, ffW1, ffb1, ffW2, ffb2):
    return pl.pallas_call(
        _lstm_body,
        out_shape=jax.ShapeDtypeStruct((BL, C), jnp.float32),
    )(xT, Wih, Whh, bih, bhh, ffW1, ffb1, ffW2, ffb2)


def _final_body(a3_ref, deg2_ref, b3_ref, clw_ref, clb_ref, lo_ref, o_ref):
    si = lax.rsqrt(jnp.clip(deg2_ref[1][:, :1], 1.0, None))
    a = (a3_ref[0] + a3_ref[1]) * si + b3_ref[...]
    mask = (lax.broadcasted_iota(jnp.int32, (NPAD, 1), 0) < N).astype(jnp.float32)
    hg = jnp.sum(jnp.maximum(a, 0.0) * mask, axis=0, keepdims=True) * (1.0 / N)
    y = jnp.dot(hg, clw_ref[...], preferred_element_type=jnp.float32) + clb_ref[...]
    o_ref[...] = lo_ref[...] + y


def _final(agg3, deg2, b3p, clwp, clb, lo):
    return pl.pallas_call(
        _final_body,
        out_shape=jax.ShapeDtypeStruct((BL, C), jnp.float32),
    )(agg3, deg2, b3p, clwp, clb, lo)


def kernel(x, W1, b1, W2, b2, W3, b3, Wih, Whh, bih, bhh, ffW1, ffb1, ffW2,
           ffb2, clW, clb, edge_index):
    f32 = jnp.float32
    x_pad = jnp.concatenate([x, jnp.zeros((NPAD - N, D), f32)], axis=0)
    pad_idx = jnp.full((EPAD - E,), N, jnp.int32)
    src2 = jnp.concatenate([edge_index[0], pad_idx]).reshape(ROWS, 128)
    dst2 = jnp.concatenate([edge_index[1], pad_idx]).reshape(ROWS, 128)
    srcdst = jnp.stack([src2, dst2])

    deg2 = _deg_kernel()(srcdst)
    xn = _prep(x_pad, deg2)
    agg1 = _agg128_kernel()(xn, src2, dst2)
    h1n = _layer1(agg1, deg2, W1, b1.reshape(1, D))
    agg2 = _agg128_kernel()(h1n, src2, dst2)
    w3p = jnp.concatenate([W3, jnp.zeros((D, 8), f32)], axis=1)
    p = _layer2(agg2, deg2, W2, b2.reshape(1, D), w3p)
    agg3 = _agg16_kernel()(p, src2, dst2)

    xT = jnp.pad(jnp.swapaxes(x.reshape(909, 11, D), 0, 1), ((0, 0), (0, BL - 909), (0, 0)))
    lo = _lstm(xT, Wih, Whh, bih.reshape(1, 4 * D), bhh.reshape(1, 4 * D),
               ffW1, ffb1.reshape(1, 64), ffW2, ffb2.reshape(1, C))

    b3p = jnp.concatenate([b3, jnp.zeros((8,), f32)]).reshape(1, 16)
    clwp = jnp.concatenate([clW, jnp.zeros((8, C), f32)], axis=0)
    out = _final(agg3, deg2, b3p, clwp, clb.reshape(1, C), lo)
    return out[:909]


# speedup vs baseline: 1.2877x; 1.2877x over previous
import functools

import jax
import jax.numpy as jnp
from jax import lax
from jax.experimental import pallas as pl
from jax.experimental.pallas import tpu as pltpu
from jax.experimental.pallas import tpu_sc as plsc

N = 9999
NPAD = 10240
E = 319968
EPAD = 327680
ROWS = EPAD // 128
D = 128
C = 2
BL = 912


@functools.lru_cache(maxsize=None)
def _mesh():
    return plsc.VectorSubcoreMesh(core_axis_name="c", subcore_axis_name="s")


def _deg_body(srcdst_hbm, out_hbm, buf, idxb, acc, sem_s0, sem_s1, sem_x):
    c = lax.axis_index("c")
    s = lax.axis_index("s")
    NT = 40
    sems = (sem_s0, sem_s1)

    def fill_zero(i, _):
        buf[i, :] = jnp.zeros((16,), jnp.float32)
        return 0

    lax.fori_loop(0, 512, fill_zero, 0)
    pltpu.sync_copy(buf, acc.at[pl.ds(s * 640, 512)])
    pltpu.sync_copy(buf.at[pl.ds(0, 128)], acc.at[pl.ds(s * 640 + 512, 128)])

    def fill_one(i, _):
        buf[i, :] = jnp.ones((16,), jnp.float32)
        return 0

    lax.fori_loop(0, 128, fill_one, 0)
    plsc.subcore_barrier()

    base = s * 160

    def issue_s(p, q):
        for j in range(4):
            pltpu.async_copy(buf.at[pl.ds(0, 128)], acc.at[idxb.at[q * 4 + j]],
                             sems[p], add=True)

    def wait_s(p, q):
        for j in range(4):
            pltpu.make_async_copy(buf.at[pl.ds(0, 128)],
                                  acc.at[idxb.at[q * 4 + j]], sems[p]).wait()

    def load_idx(q, r0):
        pltpu.async_copy(srcdst_hbm.at[c, pl.ds(r0, 4)],
                         idxb.at[pl.ds(q * 4, 4)], sem_x)

    def wait_idx(q):
        pltpu.make_async_copy(srcdst_hbm.at[c, pl.ds(0, 4)],
                              idxb.at[pl.ds(q * 4, 4)], sem_x).wait()

    pltpu.sync_copy(srcdst_hbm.at[c, pl.ds(base, 4)], idxb.at[pl.ds(0, 4)])

    def macro(m, _):
        for q in range(4):
            i = 4 * m + q

            @pl.when(i >= 2)
            def _():
                wait_s(q % 2, (q + 2) % 4)

            @pl.when(i >= 1)
            def _():
                wait_idx(q)

            issue_s(q % 2, q)

            @pl.when(i <= NT - 2)
            def _():
                load_idx((q + 1) % 4, base + (i + 1) * 4)

        return 0

    lax.fori_loop(0, NT // 4, macro, 0)
    wait_s((NT - 2) % 2, (NT - 2) % 4)
    wait_s((NT - 1) % 2, (NT - 1) % 4)
    plsc.subcore_barrier()
    pltpu.sync_copy(acc.at[pl.ds(s * 640, 640)], out_hbm.at[c, pl.ds(s * 640, 640)])


@functools.lru_cache(maxsize=None)
def _deg_kernel():
    return functools.partial(
        pl.kernel,
        compiler_params=pltpu.CompilerParams(use_tc_tiling_on_sc=False),
        out_type=jax.ShapeDtypeStruct((2, NPAD, 16), jnp.float32),
        mesh=_mesh(),
        scratch_types=[
            pltpu.VMEM((512, 16), jnp.float32),
            pltpu.VMEM((16, 128), jnp.int32),
            pltpu.VMEM_SHARED((NPAD, 16), jnp.float32),
            pltpu.SemaphoreType.DMA,
            pltpu.SemaphoreType.DMA,
            pltpu.SemaphoreType.DMA,
        ],
    )(_deg_body)


def _make_agg(width, nstreams, edge_split):
    def body(tab_hbm, src_hbm, dst_hbm, out_hbm, buf, sidx, didx, acc,
             g0, g1, g2, g3, s0, s1, s2, s3, sem_x):
        c = lax.axis_index("c")
        s = lax.axis_index("s")
        nv = width // 16
        gsems = (g0, g1, g2, g3)
        ssems = (s0, s1, s2, s3)

        def fill_zero(i, _):
            for j in range(nv):
                buf[i, pl.ds(j * 16, 16)] = jnp.zeros((16,), jnp.float32)
            return 0

        lax.fori_loop(0, 512, fill_zero, 0)
        pltpu.sync_copy(buf.at[pl.ds(0, 512)], acc.at[pl.ds(s * 640, 512)])
        pltpu.sync_copy(buf.at[pl.ds(0, 128)], acc.at[pl.ds(s * 640 + 512, 128)])
        plsc.subcore_barrier()

        if edge_split:
            base = c * (nstreams * 16) + s * nstreams
        else:
            base = s * nstreams

        def src_slice(r0, n):
            if edge_split:
                return src_hbm.at[pl.ds(r0, n)]
            return src_hbm.at[c, pl.ds(r0, n)]

        def bufsl(sl):
            return buf.at[pl.ds((sl % 8) * 128, 128)]

        def issue_g(q):
            pltpu.async_copy(tab_hbm.at[sidx.at[q % 16]], bufsl(q),
                             gsems[q % 4])

        def wait_g(q):
            pltpu.make_async_copy(tab_hbm.at[sidx.at[q % 16]], bufsl(q),
                                  gsems[q % 4]).wait()

        def issue_s(q):
            pltpu.async_copy(bufsl(q), acc.at[didx.at[q % 16]], ssems[q % 4],
                             add=True)

        def wait_s(q):
            pltpu.make_async_copy(bufsl(q), acc.at[didx.at[q % 16]],
                                  ssems[q % 4]).wait()

        def load_idx(ksl, r0):
            pltpu.async_copy(src_slice(r0, 4), sidx.at[pl.ds((ksl % 4) * 4, 4)],
                             sem_x)
            pltpu.async_copy(dst_hbm.at[pl.ds(r0, 4)],
                             didx.at[pl.ds((ksl % 4) * 4, 4)], sem_x)

        def wait_idx(ksl):
            pltpu.make_async_copy(src_slice(0, 4),
                                  sidx.at[pl.ds((ksl % 4) * 4, 4)], sem_x).wait()
            pltpu.make_async_copy(dst_hbm.at[pl.ds(0, 4)],
                                  didx.at[pl.ds((ksl % 4) * 4, 4)], sem_x).wait()

        pltpu.sync_copy(src_slice(base, 4), sidx.at[pl.ds(0, 4)])
        pltpu.sync_copy(dst_hbm.at[pl.ds(base, 4)], didx.at[pl.ds(0, 4)])
        load_idx(1, base + 4)
        for q in range(4):
            issue_g(q)

        def macro(m, _):
            for q in range(16):
                i = 16 * m + q
                wait_g(q)

                @pl.when(i >= 4)
                def _():
                    wait_s(q + 12)

                issue_s(q)

                if q % 4 == 0:
                    @pl.when(i <= nstreams - 5)
                    def _():
                        wait_idx(q // 4 + 1)

                    @pl.when(i <= nstreams - 9)
                    def _():
                        load_idx(q // 4 + 2, base + i + 8)

                @pl.when(i <= nstreams - 5)
                def _():
                    issue_g_late(q)

            return 0

        def issue_g_late(q):
            pltpu.async_copy(tab_hbm.at[sidx.at[(q + 4) % 16]], bufsl(q + 4),
                             gsems[q % 4])

        lax.fori_loop(0, nstreams // 16, macro, 0)
        for t in range(4):
            wait_s(12 + t)
        plsc.subcore_barrier()
        pltpu.sync_copy(acc.at[pl.ds(s * 640, 640)], out_hbm.at[c, pl.ds(s * 640, 640)])

    return functools.partial(
        pl.kernel,
        compiler_params=pltpu.CompilerParams(use_tc_tiling_on_sc=False),
        out_type=jax.ShapeDtypeStruct((2, NPAD, width), jnp.float32),
        mesh=_mesh(),
        scratch_types=[
            pltpu.VMEM((1024, width), jnp.float32),
            pltpu.VMEM((16, 128), jnp.int32),
            pltpu.VMEM((16, 128), jnp.int32),
            pltpu.VMEM_SHARED((NPAD, width), jnp.float32),
        ] + [pltpu.SemaphoreType.DMA] * 9,
    )(body)


@functools.lru_cache(maxsize=None)
def _aggc_kernel():
    return _make_agg(64, 160, edge_split=False)


@functools.lru_cache(maxsize=None)
def _agg16_kernel():
    return _make_agg(16, 80, edge_split=True)


_BLK = 1024


def _prep_body(x_ref, dego_ref, o_ref):
    so = lax.rsqrt(jnp.clip(dego_ref[0][:, :1], 1.0, None))
    xn = x_ref[...] * so
    o_ref[0] = xn[:, :64]
    o_ref[1] = xn[:, 64:]


def _prep(x_pad, deg2):
    return pl.pallas_call(
        _prep_body,
        grid=(NPAD // _BLK,),
        in_specs=[
            pl.BlockSpec((_BLK, D), lambda i: (i, 0)),
            pl.BlockSpec((1, _BLK, 16), lambda i: (0, i, 0)),
        ],
        out_specs=pl.BlockSpec((2, _BLK, 64), lambda i: (0, i, 0)),
        out_shape=jax.ShapeDtypeStruct((2, NPAD, 64), jnp.float32),
    )(x_pad, deg2)


def _layer1_body(a0_ref, a1_ref, degi_ref, dego_ref, w_ref, b_ref, o_ref):
    si = lax.rsqrt(jnp.clip(degi_ref[0][:, :1], 1.0, None))
    so = lax.rsqrt(jnp.clip(dego_ref[0][:, :1], 1.0, None))
    w = w_ref[...]
    t = (jnp.dot(a0_ref[0] * si, w[:64], preferred_element_type=jnp.float32)
         + jnp.dot(a1_ref[0] * si, w[64:], preferred_element_type=jnp.float32)
         + b_ref[...])
    hn = jnp.maximum(t, 0.0) * so
    o_ref[0] = hn[:, :64]
    o_ref[1] = hn[:, 64:]


def _layer2_body(a0_ref, a1_ref, degi_ref, dego_ref, w_ref, b_ref, w3_ref, o_ref):
    si = lax.rsqrt(jnp.clip(degi_ref[0][:, :1], 1.0, None))
    so = lax.rsqrt(jnp.clip(dego_ref[0][:, :1], 1.0, None))
    w = w_ref[...]
    t = (jnp.dot(a0_ref[0] * si, w[:64], preferred_element_type=jnp.float32)
         + jnp.dot(a1_ref[0] * si, w[64:], preferred_element_type=jnp.float32)
         + b_ref[...])
    hn = jnp.tanh(t) * so
    o_ref[...] = jnp.dot(hn, w3_ref[...], preferred_element_type=jnp.float32)


def _layer1(agg, deg2, w, b):
    return pl.pallas_call(
        _layer1_body,
        grid=(NPAD // _BLK,),
        in_specs=[
            pl.BlockSpec((1, _BLK, 64), lambda i: (0, i, 0)),
            pl.BlockSpec((1, _BLK, 64), lambda i: (1, i, 0)),
            pl.BlockSpec((1, _BLK, 16), lambda i: (1, i, 0)),
            pl.BlockSpec((1, _BLK, 16), lambda i: (0, i, 0)),
            pl.BlockSpec((D, D), lambda i: (0, 0)),
            pl.BlockSpec((1, D), lambda i: (0, 0)),
        ],
        out_specs=pl.BlockSpec((2, _BLK, 64), lambda i: (0, i, 0)),
        out_shape=jax.ShapeDtypeStruct((2, NPAD, 64), jnp.float32),
    )(agg, agg, deg2, deg2, w, b)


def _layer2(agg, deg2, w, b, w3p):
    return pl.pallas_call(
        _layer2_body,
        grid=(NPAD // _BLK,),
        in_specs=[
            pl.BlockSpec((1, _BLK, 64), lambda i: (0, i, 0)),
            pl.BlockSpec((1, _BLK, 64), lambda i: (1, i, 0)),
            pl.BlockSpec((1, _BLK, 16), lambda i: (1, i, 0)),
            pl.BlockSpec((1, _BLK, 16), lambda i: (0, i, 0)),
            pl.BlockSpec((D, D), lambda i: (0, 0)),
            pl.BlockSpec((1, D), lambda i: (0, 0)),
            pl.BlockSpec((D, 16), lambda i: (0, 0)),
        ],
        out_specs=pl.BlockSpec((_BLK, 16), lambda i: (i, 0)),
        out_shape=jax.ShapeDtypeStruct((NPAD, 16), jnp.float32),
    )(agg, agg, deg2, deg2, w, b, w3p)


def _lstm_body(x_ref, wih_ref, whh_ref, bih_ref, bhh_ref, fw1_ref, fb1_ref,
               fw2_ref, fb2_ref, o_ref):
    b4 = bih_ref[...] + bhh_ref[...]
    wih = wih_ref[...]
    whh = whh_ref[...]
    dn = (((1,), (1,)), ((), ()))

    def step(t, hc):
        h, cc = hc
        xt = x_ref[t]
        g = (lax.dot_general(xt, wih, dn, preferred_element_type=jnp.float32)
             + lax.dot_general(h, whh, dn, preferred_element_type=jnp.float32)
             + b4)
        ig = jax.nn.sigmoid(g[:, 0:128])
        fg = jax.nn.sigmoid(g[:, 128:256])
        gg = jnp.tanh(g[:, 256:384])
        og = jax.nn.sigmoid(g[:, 384:512])
        cc = fg * cc + ig * gg
        return (og * jnp.tanh(cc), cc)

    h0 = jnp.zeros((BL, 128), jnp.float32)
    h, _ = lax.fori_loop(0, 11, step, (h0, h0))
    t1 = jnp.dot(h, fw1_ref[...], preferred_element_type=jnp.float32) + fb1_ref[...]
    o_ref[...] = jnp.dot(t1, fw2_ref[...], preferred_element_type=jnp.float32) + fb2_ref[...]


def _lstm(xT, Wih, Whh, bih, bhh, ffW1, ffb1, ffW2, ffb2):
    return pl.pallas_call(
        _lstm_body,
        out_shape=jax.ShapeDtypeStruct((BL, C), jnp.float32),
    )(xT, Wih, Whh, bih, bhh, ffW1, ffb1, ffW2, ffb2)


def _final_body(a3_ref, deg2_ref, b3_ref, clw_ref, clb_ref, lo_ref, o_ref):
    si = lax.rsqrt(jnp.clip(deg2_ref[1][:, :1], 1.0, None))
    a = (a3_ref[0] + a3_ref[1]) * si + b3_ref[...]
    mask = (lax.broadcasted_iota(jnp.int32, (NPAD, 1), 0) < N).astype(jnp.float32)
    hg = jnp.sum(jnp.maximum(a, 0.0) * mask, axis=0, keepdims=True) * (1.0 / N)
    y = jnp.dot(hg, clw_ref[...], preferred_element_type=jnp.float32) + clb_ref[...]
    o_ref[...] = lo_ref[...] + y


def _final(agg3, deg2, b3p, clwp, clb, lo):
    return pl.pallas_call(
        _final_body,
        out_shape=jax.ShapeDtypeStruct((BL, C), jnp.float32),
    )(agg3, deg2, b3p, clwp, clb, lo)


def kernel(x, W1, b1, W2, b2, W3, b3, Wih, Whh, bih, bhh, ffW1, ffb1, ffW2,
           ffb2, clW, clb, edge_index):
    f32 = jnp.float32
    x_pad = jnp.concatenate([x, jnp.zeros((NPAD - N, D), f32)], axis=0)
    pad_idx = jnp.full((EPAD - E,), N, jnp.int32)
    src2 = jnp.concatenate([edge_index[0], pad_idx]).reshape(ROWS, 128)
    dst2 = jnp.concatenate([edge_index[1], pad_idx]).reshape(ROWS, 128)
    srcdst = jnp.stack([src2, dst2])
    srcoff = jnp.stack([src2, src2 + NPAD])

    deg2 = _deg_kernel()(srcdst)
    xn = _prep(x_pad, deg2)
    agg1 = _aggc_kernel()(xn.reshape(2 * NPAD, 64), srcoff, dst2)
    h1n = _layer1(agg1, deg2, W1, b1.reshape(1, D))
    agg2 = _aggc_kernel()(h1n.reshape(2 * NPAD, 64), srcoff, dst2)
    w3p = jnp.concatenate([W3, jnp.zeros((D, 8), f32)], axis=1)
    p = _layer2(agg2, deg2, W2, b2.reshape(1, D), w3p)
    agg3 = _agg16_kernel()(p, src2, dst2)

    xT = jnp.pad(jnp.swapaxes(x.reshape(909, 11, D), 0, 1), ((0, 0), (0, BL - 909), (0, 0)))
    lo = _lstm(xT, Wih, Whh, bih.reshape(1, 4 * D), bhh.reshape(1, 4 * D),
               ffW1, ffb1.reshape(1, 64), ffW2, ffb2.reshape(1, C))

    b3p = jnp.concatenate([b3, jnp.zeros((8,), f32)]).reshape(1, 16)
    clwp = jnp.concatenate([clW, jnp.zeros((8, C), f32)], axis=0)
    out = _final(agg3, deg2, b3p, clwp, clb.reshape(1, C), lo)
    return out[:909]
